# baseline (speedup 1.0000x reference)
import functools

import jax
import jax.numpy as jnp
import numpy as np
from jax import lax
from jax.experimental import pallas as pl
from jax.experimental.pallas import tpu as pltpu
from jax.experimental.pallas import tpu_sc as plsc

DIM = 128
HEADS = 8
DH = DIM // HEADS
N = 10000
NPAD = 10112
TWR = 80
AW = 144

NC = 2
NS = 16
NW = NC * NS
EB = 64


def _leaky(x, slope):
    return jnp.where(x >= 0, x, slope * x)


def _pack_rows(xw_lo, xw_hi, asd):
    lo = lax.bitcast_convert_type(xw_lo.astype(jnp.bfloat16), jnp.uint16).astype(jnp.uint32)
    hi = lax.bitcast_convert_type(xw_hi.astype(jnp.bfloat16), jnp.uint16).astype(jnp.uint32)
    packed = lax.bitcast_convert_type(lo | (hi << 16), jnp.float32)
    return jnp.concatenate([packed, asd], axis=1)


def _table_body(x_ref, wl_ref, wh_ref, wa_ref, t_ref, ad_ref):
    x = x_ref[...]
    xw_lo = jnp.dot(x, wl_ref[...], preferred_element_type=jnp.float32)
    xw_hi = jnp.dot(x, wh_ref[...], preferred_element_type=jnp.float32)
    asd = jnp.dot(x, wa_ref[...], preferred_element_type=jnp.float32)
    t_ref[...] = _pack_rows(xw_lo, xw_hi, asd)
    ad_ref[...] = asd


def _build_table(x, wl, wh, wa, block=1264):
    n = x.shape[0]
    return pl.pallas_call(
        _table_body,
        grid=(n // block,),
        in_specs=[
            pl.BlockSpec((block, DIM), lambda i: (i, 0)),
            pl.BlockSpec((DIM, DIM // 2), lambda i: (0, 0)),
            pl.BlockSpec((DIM, DIM // 2), lambda i: (0, 0)),
            pl.BlockSpec((DIM, 2 * HEADS), lambda i: (0, 0)),
        ],
        out_specs=[
            pl.BlockSpec((block, TWR), lambda i: (i, 0)),
            pl.BlockSpec((block, 2 * HEADS), lambda i: (i, 0)),
        ],
        out_shape=[
            jax.ShapeDtypeStruct((n, TWR), jnp.float32),
            jax.ShapeDtypeStruct((n, 2 * HEADS), jnp.float32),
        ],
    )(x, wl, wh, wa)


def _node_h(acc_ref, b_ref):
    den = acc_ref[0, :, :HEADS] + acc_ref[1, :, :HEADS]
    num = acc_ref[0, :, 2 * HEADS:] + acc_ref[1, :, 2 * HEADS:]
    den = den + (den == 0).astype(jnp.float32)
    lane = lax.broadcasted_iota(jnp.int32, (HEADS, DIM), 1)
    head = lax.broadcasted_iota(jnp.int32, (HEADS, DIM), 0)
    spread = (lane // DH == head).astype(jnp.float32)
    den128 = jnp.dot(1.0 / den, spread, preferred_element_type=jnp.float32)
    return _leaky(num * den128 + b_ref[...], 0.01)


def _finalize1_body(acc_ref, b_ref, wl_ref, wh_ref, wa_ref, t_ref, ad_ref):
    h = _node_h(acc_ref, b_ref)
    xw_lo = jnp.dot(h, wl_ref[...], preferred_element_type=jnp.float32)
    xw_hi = jnp.dot(h, wh_ref[...], preferred_element_type=jnp.float32)
    asd = jnp.dot(h, wa_ref[...], preferred_element_type=jnp.float32)
    t_ref[...] = _pack_rows(xw_lo, xw_hi, asd)
    ad_ref[...] = asd


def _finalize1(acc, b, wl, wh, wa, block=1264):
    return pl.pallas_call(
        _finalize1_body,
        grid=(NPAD // block,),
        in_specs=[
            pl.BlockSpec((2, block, AW), lambda i: (0, i, 0)),
            pl.BlockSpec((1, DIM), lambda i: (0, 0)),
            pl.BlockSpec((DIM, DIM // 2), lambda i: (0, 0)),
            pl.BlockSpec((DIM, DIM // 2), lambda i: (0, 0)),
            pl.BlockSpec((DIM, 2 * HEADS), lambda i: (0, 0)),
        ],
        out_specs=[
            pl.BlockSpec((block, TWR), lambda i: (i, 0)),
            pl.BlockSpec((block, 2 * HEADS), lambda i: (i, 0)),
        ],
        out_shape=[
            jax.ShapeDtypeStruct((NPAD, TWR), jnp.float32),
            jax.ShapeDtypeStruct((NPAD, 2 * HEADS), jnp.float32),
        ],
    )(acc, b, wl, wh, wa)


def _finalize2_body(acc_ref, b_ref, m1_ref, m1b_ref, m2_ref, sh_ref, se_ref, *, block):
    i = pl.program_id(0)
    h = _node_h(acc_ref, b_ref)
    hid = _leaky(jnp.dot(h, m1_ref[...], preferred_element_type=jnp.float32)
                 + m1b_ref[...], 0.01)
    s = jnp.dot(hid, m2_ref[...], preferred_element_type=jnp.float32)
    row = i * block + lax.broadcasted_iota(jnp.int32, (block, 1), 0)
    e = jnp.where(row < N, jnp.exp(s), 0.0)

    @pl.when(i == 0)
    def _():
        sh_ref[...] = jnp.zeros_like(sh_ref)
        se_ref[...] = jnp.zeros_like(se_ref)

    sh_ref[...] += jnp.sum(e * h, axis=0, keepdims=True)
    se_ref[...] += jnp.sum(e, axis=0, keepdims=True)


def _finalize2(acc, b, m1_w, m1_b, m2_w, block=1264):
    return pl.pallas_call(
        functools.partial(_finalize2_body, block=block),
        grid=(NPAD // block,),
        in_specs=[
            pl.BlockSpec((2, block, AW), lambda i: (0, i, 0)),
            pl.BlockSpec((1, DIM), lambda i: (0, 0)),
            pl.BlockSpec((DIM, DIM // 2), lambda i: (0, 0)),
            pl.BlockSpec((1, DIM // 2), lambda i: (0, 0)),
            pl.BlockSpec((DIM // 2, 1), lambda i: (0, 0)),
        ],
        out_specs=[
            pl.BlockSpec((1, DIM), lambda i: (0, 0)),
            pl.BlockSpec((1, 1), lambda i: (0, 0)),
        ],
        out_shape=[
            jax.ShapeDtypeStruct((1, DIM), jnp.float32),
            jax.ShapeDtypeStruct((1, 1), jnp.float32),
        ],
    )(acc, b, m1_w, m1_b, m2_w)


def _head_body(sh_ref, se_ref, g1_ref, g1b_ref, g2_ref, g2b_ref, lng_ref, lnb_ref, o_ref):
    agg = sh_ref[...] / se_ref[0, 0]
    a1 = _leaky(jnp.dot(agg, g1_ref[...], preferred_element_type=jnp.float32)
                + g1b_ref[...], 0.01)
    a2 = jnp.dot(a1, g2_ref[...], preferred_element_type=jnp.float32) + g2b_ref[...]
    mu = jnp.mean(a2)
    var = jnp.mean((a2 - mu) ** 2)
    o_ref[...] = (a2 - mu) / jnp.sqrt(var + 1e-5) * lng_ref[...] + lnb_ref[...]


def _head(sh, se, g1_w, g1_b, g2_w, g2_b, ln_g, ln_b):
    return pl.pallas_call(
        _head_body,
        out_shape=jax.ShapeDtypeStruct((1, DIM), jnp.float32),
    )(sh, se, g1_w, g1_b.reshape(1, -1), g2_w, g2_b.reshape(1, -1),
      ln_g.reshape(1, -1), ln_b.reshape(1, -1))


def _bcast_lane(v, k):
    idx = jnp.full((16, 1), k, jnp.int32)
    return lax.gather(
        v, idx,
        lax.GatherDimensionNumbers(offset_dims=(), collapsed_slice_dims=(0,),
                                   start_index_map=(0,)),
        (1,), mode=lax.GatherScatterMode.PROMISE_IN_BOUNDS)


def _sc_edge_kernel(epad):
    chunk = epad // NW
    nblk = chunk // EB
    npairs = nblk // 2
    mesh = plsc.VectorSubcoreMesh(core_axis_name="c", subcore_axis_name="s",
                                  num_cores=NC, num_subcores=NS)

    @functools.partial(
        pl.kernel,
        out_type=jax.ShapeDtypeStruct((NC, NPAD, AW), jnp.float32),
        mesh=mesh,
        compiler_params=pltpu.CompilerParams(use_tc_tiling_on_sc=False,
                                             needs_layout_passes=False),
        scratch_types=[
            pltpu.VMEM_SHARED((NPAD, AW), jnp.float32),
            [pltpu.VMEM((EB,), jnp.int32)] * 2,
            [pltpu.VMEM((EB,), jnp.int32)] * 2,
            [pltpu.VMEM((EB, TWR), jnp.float32)] * 2,
            [pltpu.VMEM((EB, 2 * HEADS), jnp.float32)] * 2,
            [pltpu.VMEM((EB, AW), jnp.float32)] * 2,
            [pltpu.VMEM((EB,), jnp.int32)] * 2,
            [pltpu.SemaphoreType.DMA] * 2,
            [pltpu.SemaphoreType.DMA] * 2,
            [pltpu.SemaphoreType.DMA] * 2,
            [pltpu.SemaphoreType.DMA] * 2,
            [pltpu.SemaphoreType.DMA] * 2,
        ],
    )
    def edge_kernel(table_hbm, ad_hbm, src_hbm, dst_hbm, zeros_hbm, out_hbm,
                    acc_sh, src_v, dst_v, rows_v, ad_v, msg_v, sdst_v,
                    sem_g, sem_a, sem_s, sem_i, sem_j):
        cid = lax.axis_index("c")
        sid = lax.axis_index("s")
        wid = sid * NC + cid

        zrows = NPAD // NS
        pltpu.sync_copy(zeros_hbm.at[pl.ds(sid * zrows, zrows)],
                        acc_sh.at[pl.ds(sid * zrows, zrows)])
        plsc.subcore_barrier()

        lanes = lax.iota(jnp.int32, 16)
        leq = [lanes == h for h in range(1, HEADS)]

        def fetch_idx(blk, p):
            ebase = wid * chunk + blk * EB
            pltpu.async_copy(src_hbm.at[pl.ds(ebase, EB)], src_v[p], sem_i[p])
            pltpu.async_copy(dst_hbm.at[pl.ds(ebase, EB)], dst_v[p], sem_j[p])

        def wait_idx(p):
            pltpu.make_async_copy(src_hbm.at[pl.ds(0, EB)], src_v[p], sem_i[p]).wait()
            pltpu.make_async_copy(dst_hbm.at[pl.ds(0, EB)], dst_v[p], sem_j[p]).wait()

        def gathers(p):
            pltpu.async_copy(table_hbm.at[src_v[p]], rows_v[p], sem_g[p])
            pltpu.async_copy(ad_hbm.at[dst_v[p]], ad_v[p], sem_a[p])

        def wait_gather(p):
            pltpu.make_async_copy(table_hbm.at[src_v[p]], rows_v[p], sem_g[p]).wait()
            pltpu.make_async_copy(ad_hbm.at[dst_v[p]], ad_v[p], sem_a[p]).wait()

        def wait_scatter(p):
            pltpu.make_async_copy(msg_v[p], acc_sh.at[sdst_v[p]], sem_s[p]).wait()

        def compute(p):
            def group_body(g, c2):
                eidx = lanes + g * 16
                ws = []
                for h in range(HEADS):
                    a_s = plsc.load_gather(
                        rows_v[p], [eidx, jnp.full((16,), DIM // 2 + h, jnp.int32)])
                    a_d = plsc.load_gather(
                        ad_v[p], [eidx, jnp.full((16,), HEADS + h, jnp.int32)])
                    al = a_s + a_d
                    al = jnp.where(al >= 0, al, 0.2 * al)
                    ws.append(jnp.exp(al))
                for k in range(16):
                    e = g * 16 + k
                    wbs = [_bcast_lane(w, k) for w in ws]
                    denv = wbs[0]
                    for h in range(1, HEADS):
                        denv = jnp.where(leq[h - 1], wbs[h], denv)
                    msg_v[p][e, pl.ds(0, 16)] = denv
                    for q in range(4):
                        pw = plsc.bitcast(rows_v[p][e, pl.ds(q * 16, 16)], jnp.int32)
                        a_ = plsc.bitcast(pw << 16, jnp.float32)
                        b_ = plsc.bitcast(pw & jnp.int32(-65536), jnp.float32)
                        msg_v[p][e, pl.ds(16 + 32 * q, 16)] = a_ * wbs[2 * q]
                        msg_v[p][e, pl.ds(32 + 32 * q, 16)] = b_ * wbs[2 * q + 1]
                return c2

            lax.fori_loop(0, EB // 16, group_body, 0)
            pltpu.async_copy(msg_v[p], acc_sh.at[sdst_v[p]], sem_s[p], add=True)

        def sub_body(t, blk, p):
            if p == 0:
                wait_idx(1)
                gathers(1)
            else:
                @pl.when(t < npairs - 1)
                def _():
                    wait_idx(0)
                    gathers(0)
            wait_gather(p)

            @pl.when(t > 0)
            def _():
                wait_scatter(p)
            for c0 in range(0, EB, 16):
                sdst_v[p][pl.ds(c0, 16)] = dst_v[p][pl.ds(c0, 16)]

            @pl.when(t < npairs - 1)
            def _():
                fetch_idx(blk + 2, p)
            compute(p)

        fetch_idx(0, 0)
        fetch_idx(1, 1)
        wait_idx(0)
        gathers(0)

        def pair_body(t, carry):
            sub_body(t, 2 * t, 0)
            sub_body(t, 2 * t + 1, 1)
            return carry

        lax.fori_loop(0, npairs, pair_body, 0)
        wait_scatter(0)
        wait_scatter(1)

        plsc.subcore_barrier()
        pltpu.sync_copy(acc_sh.at[pl.ds(sid * zrows, zrows)],
                        out_hbm.at[cid, pl.ds(sid * zrows, zrows)])

    return edge_kernel


def _att_cat(W, att_src, att_dst):
    a = jnp.zeros((DIM, 2 * HEADS), jnp.float32)
    hs = jnp.arange(DIM) // DH
    ds = jnp.arange(DIM) % DH
    a = a.at[jnp.arange(DIM), hs].set(att_src[hs, ds])
    a = a.at[jnp.arange(DIM), HEADS + hs].set(att_dst[hs, ds])
    return jnp.concatenate([W, W @ a], axis=1)


_IDX_LO = np.array([32 * (j // 16) + j % 16 for j in range(DIM // 2)])


def _split_wcat(wcat):
    return wcat[:, _IDX_LO], wcat[:, _IDX_LO + DH], wcat[:, DIM:]


def kernel(features, edge_index, W1, att_src1, att_dst1, b1, W2, att_src2, att_dst2, b2, m1_w, m1_b, m2_w, m2_b, g1_w, g1_b, g2_w, g2_b, ln_g, ln_b):
    ne = edge_index.shape[1] + N
    epad = ((ne + NW * EB - 1) // (NW * EB)) * (NW * EB)
    loop = jnp.arange(N, dtype=jnp.int32)
    dummy = jnp.full((epad - ne,), N, jnp.int32)
    src = jnp.concatenate([edge_index[0].astype(jnp.int32), loop, dummy])
    dst = jnp.concatenate([edge_index[1].astype(jnp.int32), loop, dummy])
    x = jnp.pad(features, ((0, NPAD - N), (0, 0)))
    zeros_tab = jnp.zeros((NPAD, AW), jnp.float32)

    edge_sc = _sc_edge_kernel(epad)

    wl1, wh1, wa1 = _split_wcat(_att_cat(W1, att_src1, att_dst1))
    table1, ad1 = _build_table(x, wl1, wh1, wa1)
    acc1 = edge_sc(table1, ad1, src, dst, zeros_tab)

    wl2, wh2, wa2 = _split_wcat(_att_cat(W2, att_src2, att_dst2))
    table2, ad2 = _finalize1(acc1, b1.reshape(1, -1), wl2, wh2, wa2)
    acc2 = edge_sc(table2, ad2, src, dst, zeros_tab)

    sh, se = _finalize2(acc2, b2.reshape(1, -1), m1_w, m1_b.reshape(1, -1), m2_w)

    out = _head(sh, se, g1_w, g1_b, g2_w, g2_b, ln_g, ln_b)
    return out.reshape(DIM)


# speedup vs baseline: 1.4421x; 1.4421x over previous
import functools

import jax
import jax.numpy as jnp
import numpy as np
from jax import lax
from jax.experimental import pallas as pl
from jax.experimental.pallas import tpu as pltpu
from jax.experimental.pallas import tpu_sc as plsc

DIM = 128
HEADS = 8
DH = DIM // HEADS
N = 10000
NPAD = 10112
TWR = 80
AW = 144

NC = 2
NS = 16
NW = NC * NS
EB = 64


def _leaky(x, slope):
    return jnp.where(x >= 0, x, slope * x)


def _pack_rows(xw_lo, xw_hi, asd):
    lo = lax.bitcast_convert_type(xw_lo.astype(jnp.bfloat16), jnp.uint16).astype(jnp.uint32)
    hi = lax.bitcast_convert_type(xw_hi.astype(jnp.bfloat16), jnp.uint16).astype(jnp.uint32)
    packed = lax.bitcast_convert_type(lo | (hi << 16), jnp.float32)
    return jnp.concatenate([packed, asd], axis=1)


def _table_body(x_ref, wl_ref, wh_ref, wa_ref, t_ref, ad_ref):
    x = x_ref[...]
    xw_lo = jnp.dot(x, wl_ref[...], preferred_element_type=jnp.float32)
    xw_hi = jnp.dot(x, wh_ref[...], preferred_element_type=jnp.float32)
    asd = jnp.dot(x, wa_ref[...], preferred_element_type=jnp.float32)
    t_ref[...] = _pack_rows(xw_lo, xw_hi, asd)
    ad_ref[...] = asd


def _build_table(x, wl, wh, wa, block=1264):
    n = x.shape[0]
    return pl.pallas_call(
        _table_body,
        grid=(n // block,),
        in_specs=[
            pl.BlockSpec((block, DIM), lambda i: (i, 0)),
            pl.BlockSpec((DIM, DIM // 2), lambda i: (0, 0)),
            pl.BlockSpec((DIM, DIM // 2), lambda i: (0, 0)),
            pl.BlockSpec((DIM, 2 * HEADS), lambda i: (0, 0)),
        ],
        out_specs=[
            pl.BlockSpec((block, TWR), lambda i: (i, 0)),
            pl.BlockSpec((block, 2 * HEADS), lambda i: (i, 0)),
        ],
        out_shape=[
            jax.ShapeDtypeStruct((n, TWR), jnp.float32),
            jax.ShapeDtypeStruct((n, 2 * HEADS), jnp.float32),
        ],
    )(x, wl, wh, wa)


def _node_h(acc_ref, b_ref):
    den = acc_ref[0, :, :HEADS] + acc_ref[1, :, :HEADS]
    num = acc_ref[0, :, 2 * HEADS:] + acc_ref[1, :, 2 * HEADS:]
    den = den + (den == 0).astype(jnp.float32)
    lane = lax.broadcasted_iota(jnp.int32, (HEADS, DIM), 1)
    head = lax.broadcasted_iota(jnp.int32, (HEADS, DIM), 0)
    spread = (lane // DH == head).astype(jnp.float32)
    den128 = jnp.dot(1.0 / den, spread, preferred_element_type=jnp.float32)
    return _leaky(num * den128 + b_ref[...], 0.01)


def _finalize1_body(acc_ref, b_ref, wl_ref, wh_ref, wa_ref, t_ref, ad_ref):
    h = _node_h(acc_ref, b_ref)
    xw_lo = jnp.dot(h, wl_ref[...], preferred_element_type=jnp.float32)
    xw_hi = jnp.dot(h, wh_ref[...], preferred_element_type=jnp.float32)
    asd = jnp.dot(h, wa_ref[...], preferred_element_type=jnp.float32)
    t_ref[...] = _pack_rows(xw_lo, xw_hi, asd)
    ad_ref[...] = asd


def _finalize1(acc, b, wl, wh, wa, block=1264):
    return pl.pallas_call(
        _finalize1_body,
        grid=(NPAD // block,),
        in_specs=[
            pl.BlockSpec((2, block, AW), lambda i: (0, i, 0)),
            pl.BlockSpec((1, DIM), lambda i: (0, 0)),
            pl.BlockSpec((DIM, DIM // 2), lambda i: (0, 0)),
            pl.BlockSpec((DIM, DIM // 2), lambda i: (0, 0)),
            pl.BlockSpec((DIM, 2 * HEADS), lambda i: (0, 0)),
        ],
        out_specs=[
            pl.BlockSpec((block, TWR), lambda i: (i, 0)),
            pl.BlockSpec((block, 2 * HEADS), lambda i: (i, 0)),
        ],
        out_shape=[
            jax.ShapeDtypeStruct((NPAD, TWR), jnp.float32),
            jax.ShapeDtypeStruct((NPAD, 2 * HEADS), jnp.float32),
        ],
    )(acc, b, wl, wh, wa)


def _finalize2_body(acc_ref, b_ref, m1_ref, m1b_ref, m2_ref, sh_ref, se_ref, *, block):
    i = pl.program_id(0)
    h = _node_h(acc_ref, b_ref)
    hid = _leaky(jnp.dot(h, m1_ref[...], preferred_element_type=jnp.float32)
                 + m1b_ref[...], 0.01)
    s = jnp.dot(hid, m2_ref[...], preferred_element_type=jnp.float32)
    row = i * block + lax.broadcasted_iota(jnp.int32, (block, 1), 0)
    e = jnp.where(row < N, jnp.exp(s), 0.0)

    @pl.when(i == 0)
    def _():
        sh_ref[...] = jnp.zeros_like(sh_ref)
        se_ref[...] = jnp.zeros_like(se_ref)

    sh_ref[...] += jnp.sum(e * h, axis=0, keepdims=True)
    se_ref[...] += jnp.sum(e, axis=0, keepdims=True)


def _finalize2(acc, b, m1_w, m1_b, m2_w, block=1264):
    return pl.pallas_call(
        functools.partial(_finalize2_body, block=block),
        grid=(NPAD // block,),
        in_specs=[
            pl.BlockSpec((2, block, AW), lambda i: (0, i, 0)),
            pl.BlockSpec((1, DIM), lambda i: (0, 0)),
            pl.BlockSpec((DIM, DIM // 2), lambda i: (0, 0)),
            pl.BlockSpec((1, DIM // 2), lambda i: (0, 0)),
            pl.BlockSpec((DIM // 2, 1), lambda i: (0, 0)),
        ],
        out_specs=[
            pl.BlockSpec((1, DIM), lambda i: (0, 0)),
            pl.BlockSpec((1, 1), lambda i: (0, 0)),
        ],
        out_shape=[
            jax.ShapeDtypeStruct((1, DIM), jnp.float32),
            jax.ShapeDtypeStruct((1, 1), jnp.float32),
        ],
    )(acc, b, m1_w, m1_b, m2_w)


def _head_body(sh_ref, se_ref, g1_ref, g1b_ref, g2_ref, g2b_ref, lng_ref, lnb_ref, o_ref):
    agg = sh_ref[...] / se_ref[0, 0]
    a1 = _leaky(jnp.dot(agg, g1_ref[...], preferred_element_type=jnp.float32)
                + g1b_ref[...], 0.01)
    a2 = jnp.dot(a1, g2_ref[...], preferred_element_type=jnp.float32) + g2b_ref[...]
    mu = jnp.mean(a2)
    var = jnp.mean((a2 - mu) ** 2)
    o_ref[...] = (a2 - mu) / jnp.sqrt(var + 1e-5) * lng_ref[...] + lnb_ref[...]


def _head(sh, se, g1_w, g1_b, g2_w, g2_b, ln_g, ln_b):
    return pl.pallas_call(
        _head_body,
        out_shape=jax.ShapeDtypeStruct((1, DIM), jnp.float32),
    )(sh, se, g1_w, g1_b.reshape(1, -1), g2_w, g2_b.reshape(1, -1),
      ln_g.reshape(1, -1), ln_b.reshape(1, -1))


def _bcast_lane(v, k):
    idx = jnp.full((16, 1), k, jnp.int32)
    return lax.gather(
        v, idx,
        lax.GatherDimensionNumbers(offset_dims=(), collapsed_slice_dims=(0,),
                                   start_index_map=(0,)),
        (1,), mode=lax.GatherScatterMode.PROMISE_IN_BOUNDS)


def _sc_edge_kernel(epad):
    chunk = epad // NW
    nblk = chunk // EB
    npairs = nblk // 2
    mesh = plsc.VectorSubcoreMesh(core_axis_name="c", subcore_axis_name="s",
                                  num_cores=NC, num_subcores=NS)

    @functools.partial(
        pl.kernel,
        out_type=jax.ShapeDtypeStruct((NC, NPAD, AW), jnp.float32),
        mesh=mesh,
        compiler_params=pltpu.CompilerParams(use_tc_tiling_on_sc=False,
                                             needs_layout_passes=False),
        scratch_types=[
            pltpu.VMEM_SHARED((NPAD, AW), jnp.float32),
            [pltpu.VMEM((EB,), jnp.int32)] * 2,
            [pltpu.VMEM((EB,), jnp.int32)] * 2,
            [pltpu.VMEM((EB, TWR), jnp.float32)] * 2,
            [pltpu.VMEM((EB, 2 * HEADS), jnp.float32)] * 2,
            [pltpu.VMEM((EB, AW), jnp.float32)] * 2,
            [pltpu.VMEM((EB,), jnp.int32)] * 2,
            [pltpu.SemaphoreType.DMA] * 2,
            [pltpu.SemaphoreType.DMA] * 2,
            [pltpu.SemaphoreType.DMA] * 2,
            [pltpu.SemaphoreType.DMA] * 2,
            [pltpu.SemaphoreType.DMA] * 2,
        ],
    )
    def edge_kernel(table_hbm, ad_hbm, src_hbm, dst_hbm, zeros_hbm, out_hbm,
                    acc_sh, src_v, dst_v, rows_v, ad_v, msg_v, sdst_v,
                    sem_g, sem_a, sem_s, sem_i, sem_j):
        cid = lax.axis_index("c")
        sid = lax.axis_index("s")
        wid = sid * NC + cid

        zrows = NPAD // NS
        pltpu.sync_copy(zeros_hbm.at[pl.ds(sid * zrows, zrows)],
                        acc_sh.at[pl.ds(sid * zrows, zrows)])
        plsc.subcore_barrier()

        lanes = lax.iota(jnp.int32, 16)
        leq = [lanes == h for h in range(1, HEADS)]

        def fetch_idx(blk, p):
            ebase = wid * chunk + blk * EB
            pltpu.async_copy(src_hbm.at[pl.ds(ebase, EB)], src_v[p], sem_i[p])
            pltpu.async_copy(dst_hbm.at[pl.ds(ebase, EB)], dst_v[p], sem_j[p])

        def wait_idx(p):
            pltpu.make_async_copy(src_hbm.at[pl.ds(0, EB)], src_v[p], sem_i[p]).wait()
            pltpu.make_async_copy(dst_hbm.at[pl.ds(0, EB)], dst_v[p], sem_j[p]).wait()

        def gathers(p):
            pltpu.async_copy(table_hbm.at[src_v[p]], rows_v[p], sem_g[p])
            pltpu.async_copy(ad_hbm.at[dst_v[p]], ad_v[p], sem_a[p])

        def wait_gather(p):
            pltpu.make_async_copy(table_hbm.at[src_v[p]], rows_v[p], sem_g[p]).wait()
            pltpu.make_async_copy(ad_hbm.at[dst_v[p]], ad_v[p], sem_a[p]).wait()

        def wait_scatter(p):
            pltpu.make_async_copy(msg_v[p], acc_sh.at[sdst_v[p]], sem_s[p]).wait()

        def compute(p):
            def group_body(g, c2):
                eidx = lanes + g * 16
                ws = []
                for h in range(HEADS):
                    a_s = plsc.load_gather(
                        rows_v[p], [eidx, jnp.full((16,), DIM // 2 + h, jnp.int32)])
                    a_d = plsc.load_gather(
                        ad_v[p], [eidx, jnp.full((16,), HEADS + h, jnp.int32)])
                    al = a_s + a_d
                    al = jnp.where(al >= 0, al, 0.2 * al)
                    ws.append(jnp.exp(al))
                for k in range(16):
                    e = g * 16 + k
                    pws = [plsc.bitcast(rows_v[p][e, pl.ds(q * 16, 16)], jnp.int32)
                           for q in range(4)]
                    wbs = [_bcast_lane(w, k) for w in ws]
                    denv = wbs[0]
                    for h in range(1, HEADS):
                        denv = jnp.where(leq[h - 1], wbs[h], denv)
                    prods = []
                    for q in range(4):
                        a_ = plsc.bitcast(pws[q] << 16, jnp.float32)
                        b_ = plsc.bitcast(pws[q] & jnp.int32(-65536), jnp.float32)
                        prods.append(a_ * wbs[2 * q])
                        prods.append(b_ * wbs[2 * q + 1])
                    msg_v[p][e, pl.ds(0, 16)] = denv
                    for q in range(8):
                        msg_v[p][e, pl.ds(16 + 16 * q, 16)] = prods[q]
                return c2

            lax.fori_loop(0, EB // 16, group_body, 0)
            pltpu.async_copy(msg_v[p], acc_sh.at[sdst_v[p]], sem_s[p], add=True)

        def sub_body(t, blk, p):
            if p == 0:
                wait_idx(1)
                gathers(1)
            else:
                @pl.when(t < npairs - 1)
                def _():
                    wait_idx(0)
                    gathers(0)
            wait_gather(p)

            @pl.when(t > 0)
            def _():
                wait_scatter(p)
            for c0 in range(0, EB, 16):
                sdst_v[p][pl.ds(c0, 16)] = dst_v[p][pl.ds(c0, 16)]

            @pl.when(t < npairs - 1)
            def _():
                fetch_idx(blk + 2, p)
            compute(p)

        fetch_idx(0, 0)
        fetch_idx(1, 1)
        wait_idx(0)
        gathers(0)

        def pair_body(t, carry):
            sub_body(t, 2 * t, 0)
            sub_body(t, 2 * t + 1, 1)
            return carry

        lax.fori_loop(0, npairs, pair_body, 0)
        wait_scatter(0)
        wait_scatter(1)

        plsc.subcore_barrier()
        pltpu.sync_copy(acc_sh.at[pl.ds(sid * zrows, zrows)],
                        out_hbm.at[cid, pl.ds(sid * zrows, zrows)])

    return edge_kernel


def _att_cat(W, att_src, att_dst):
    a = jnp.zeros((DIM, 2 * HEADS), jnp.float32)
    hs = jnp.arange(DIM) // DH
    ds = jnp.arange(DIM) % DH
    a = a.at[jnp.arange(DIM), hs].set(att_src[hs, ds])
    a = a.at[jnp.arange(DIM), HEADS + hs].set(att_dst[hs, ds])
    return jnp.concatenate([W, W @ a], axis=1)


_IDX_LO = np.array([32 * (j // 16) + j % 16 for j in range(DIM // 2)])


def _split_wcat(wcat):
    return wcat[:, _IDX_LO], wcat[:, _IDX_LO + DH], wcat[:, DIM:]


def kernel(features, edge_index, W1, att_src1, att_dst1, b1, W2, att_src2, att_dst2, b2, m1_w, m1_b, m2_w, m2_b, g1_w, g1_b, g2_w, g2_b, ln_g, ln_b):
    ne = edge_index.shape[1] + N
    epad = ((ne + NW * EB - 1) // (NW * EB)) * (NW * EB)
    loop = jnp.arange(N, dtype=jnp.int32)
    dummy = jnp.full((epad - ne,), N, jnp.int32)
    src = jnp.concatenate([edge_index[0].astype(jnp.int32), loop, dummy])
    dst = jnp.concatenate([edge_index[1].astype(jnp.int32), loop, dummy])
    x = jnp.pad(features, ((0, NPAD - N), (0, 0)))
    zeros_tab = jnp.zeros((NPAD, AW), jnp.float32)

    edge_sc = _sc_edge_kernel(epad)

    wl1, wh1, wa1 = _split_wcat(_att_cat(W1, att_src1, att_dst1))
    table1, ad1 = _build_table(x, wl1, wh1, wa1)
    acc1 = edge_sc(table1, ad1, src, dst, zeros_tab)

    wl2, wh2, wa2 = _split_wcat(_att_cat(W2, att_src2, att_dst2))
    table2, ad2 = _finalize1(acc1, b1.reshape(1, -1), wl2, wh2, wa2)
    acc2 = edge_sc(table2, ad2, src, dst, zeros_tab)

    sh, se = _finalize2(acc2, b2.reshape(1, -1), m1_w, m1_b.reshape(1, -1), m2_w)

    out = _head(sh, se, g1_w, g1_b, g2_w, g2_b, ln_g, ln_b)
    return out.reshape(DIM)
